# Initial kernel scaffold; baseline (speedup 1.0000x reference)
import jax
import jax.numpy as jnp
from jax.experimental import pallas as pl


def kernel(x, edge_index, W1, b1, W2, b2):
    raise NotImplementedError("write your pallas kernel here")


# speedup vs baseline: 44.0105x; 44.0105x over previous
import functools

import jax
import jax.numpy as jnp
from jax import lax
from jax.experimental import pallas as pl
from jax.experimental.pallas import tpu as pltpu
import jax.experimental.pallas.tpu_sc as plsc

N = 100000
E = 3200000
D = 128
H = 16

NC = 2
NS = 16
NW = NC * NS
G = 128
GPC = 8
CPT = 98
GPT = GPC * CPT
E_PAD = NW * GPT * G
BN = 2048
N_PAD = 49 * BN
R_TILE = N_PAD // NS

@functools.lru_cache(maxsize=None)
def _make_sc_pass(gather_src):
    mesh = plsc.VectorSubcoreMesh(core_axis_name="c", subcore_axis_name="s",
                                  num_cores=NC, num_subcores=NS)
    scratch = [
        pltpu.VMEM((GPC, G), jnp.int32),
        pltpu.VMEM((GPC, G), jnp.int32),
        pltpu.VMEM((GPC, G, H), jnp.float32),
        pltpu.VMEM_SHARED((N_PAD, H), jnp.float32),
        pltpu.SemaphoreType.DMA,
    ]

    @functools.partial(
        pl.kernel,
        mesh=mesh,
        out_type=jax.ShapeDtypeStruct((NC, N_PAD, H), jnp.float32),
        scratch_types=scratch,
        compiler_params=pltpu.CompilerParams(use_tc_tiling_on_sc=False),
    )
    def sc_pass(row2d, col2d, src_hbm, zeros_hbm, out_hbm,
                rowv, colv, gath, accum, sem):
        cid = lax.axis_index("c")
        sid = lax.axis_index("s")
        wid = sid * NC + cid

        pltpu.sync_copy(zeros_hbm.at[pl.ds(sid * R_TILE, R_TILE)],
                        accum.at[pl.ds(sid * R_TILE, R_TILE)])
        if not gather_src:
            pltpu.sync_copy(src_hbm.at[pl.ds(0, G)], gath.at[0])
        plsc.subcore_barrier()

        def chunk(ci, carry):
            gbase = wid * GPT + ci * GPC
            pltpu.sync_copy(row2d.at[pl.ds(gbase, GPC)], rowv)
            if gather_src:
                pltpu.sync_copy(col2d.at[pl.ds(gbase, GPC)], colv)
                descs = [pltpu.async_copy(src_hbm.at[rowv.at[j]],
                                          gath.at[j], sem)
                         for j in range(GPC)]
                for d in descs:
                    d.wait()
                for j in range(GPC):
                    pltpu.sync_copy(gath.at[j], accum.at[colv.at[j]], add=True)
            else:
                for j in range(GPC):
                    pltpu.sync_copy(gath.at[0], accum.at[rowv.at[j]], add=True)
            return carry

        lax.fori_loop(0, CPT, chunk, 0)
        plsc.subcore_barrier()
        pltpu.sync_copy(accum.at[pl.ds(sid * R_TILE, R_TILE)],
                        out_hbm.at[cid, pl.ds(sid * R_TILE, R_TILE)])

    return sc_pass


def _tc1_body(x_ref, w_ref, b_ref, d0_ref, d1_ref, hs_ref, dinv_ref):
    i = pl.program_id(0)
    deg = d0_ref[...] + d1_ref[...] + 1.0
    rid = lax.broadcasted_iota(jnp.int32, (BN, H), 0) + i * BN
    dinv = jnp.where(rid < N, lax.rsqrt(deg), 0.0)
    h = jnp.dot(x_ref[...], w_ref[...], preferred_element_type=jnp.float32)
    hs_ref[...] = dinv * (h + b_ref[...])
    dinv_ref[...] = dinv


def _tc2_body(a0_ref, a1_ref, hs_ref, dinv_ref, w_ref, b_ref, out_ref):
    dinv = dinv_ref[...]
    z = jnp.maximum(dinv * (a0_ref[...] + a1_ref[...] + hs_ref[...]), 0.0)
    h = jnp.dot(z, w_ref[...], preferred_element_type=jnp.float32)
    out_ref[...] = dinv * (h + b_ref[...])


def _tc3_body(a0_ref, a1_ref, hs_ref, dinv_ref, out_ref):
    o = dinv_ref[...] * (a0_ref[...] + a1_ref[...] + hs_ref[...])
    m = jnp.max(o, axis=1, keepdims=True)
    e = jnp.exp(o - m)
    s = jnp.sum(e, axis=1, keepdims=True)
    out_ref[...] = (o - m) - jnp.log(s)


def _row_spec():
    return pl.BlockSpec((BN, H), lambda i: (i, 0))


def _const_spec(shape):
    return pl.BlockSpec(shape, lambda i: (0, 0))


_GRID = N_PAD // BN

_tc1 = pl.pallas_call(
    _tc1_body,
    grid=(_GRID,),
    in_specs=[pl.BlockSpec((BN, D), lambda i: (i, 0)),
              _const_spec((D, H)), _const_spec((1, H)),
              _row_spec(), _row_spec()],
    out_specs=[_row_spec(), _row_spec()],
    out_shape=[jax.ShapeDtypeStruct((N_PAD, H), jnp.float32),
               jax.ShapeDtypeStruct((N_PAD, H), jnp.float32)],
)

_tc2 = pl.pallas_call(
    _tc2_body,
    grid=(_GRID,),
    in_specs=[_row_spec(), _row_spec(), _row_spec(), _row_spec(),
              _const_spec((H, H)), _const_spec((1, H))],
    out_specs=_row_spec(),
    out_shape=jax.ShapeDtypeStruct((N_PAD, H), jnp.float32),
)

_tc3 = pl.pallas_call(
    _tc3_body,
    grid=(_GRID,),
    in_specs=[_row_spec(), _row_spec(), _row_spec(), _row_spec()],
    out_specs=_row_spec(),
    out_shape=jax.ShapeDtypeStruct((N_PAD, H), jnp.float32),
)


def kernel(x, edge_index, W1, b1, W2, b2):
    pad_e = E_PAD - E
    row2d = jnp.concatenate(
        [edge_index[0], jnp.full((pad_e,), N, jnp.int32)]).reshape(E_PAD // G, G)
    col2d = jnp.concatenate(
        [edge_index[1], jnp.full((pad_e,), N, jnp.int32)]).reshape(E_PAD // G, G)

    zeros = jnp.zeros((N_PAD, H), jnp.float32)
    ones_blk = jnp.ones((G, H), jnp.float32)
    x_pad = jnp.concatenate([x, jnp.zeros((N_PAD - N, D), jnp.float32)], axis=0)
    w1t = W1.T
    w2t = W2.T
    b1r = b1.reshape(1, H)
    b2r = b2.reshape(1, H)

    sc_degree = _make_sc_pass(False)
    sc_propagate = _make_sc_pass(True)

    degp = sc_degree(row2d, col2d, ones_blk, zeros)
    hs1, dinv = _tc1(x_pad, w1t, b1r, degp[0], degp[1])
    acc1 = sc_propagate(row2d, col2d, hs1, zeros)
    hs2 = _tc2(acc1[0], acc1[1], hs1, dinv, w2t, b2r)
    acc2 = sc_propagate(row2d, col2d, hs2, zeros)
    out = _tc3(acc2[0], acc2[1], hs2, dinv)
    return out[:N]


# speedup vs baseline: 50.1310x; 1.1391x over previous
import functools

import jax
import jax.numpy as jnp
from jax import lax
from jax.experimental import pallas as pl
from jax.experimental.pallas import tpu as pltpu
import jax.experimental.pallas.tpu_sc as plsc

N = 100000
E = 3200000
D = 128
H = 16

NC = 2
NS = 16
NW = NC * NS
G = 128
GPC = 6
CPT = 131
GPT = GPC * CPT
E_PAD = NW * GPT * G
BN = 2048
N_PAD = 49 * BN
R_TILE = N_PAD // NS

@functools.lru_cache(maxsize=None)
def _make_sc_pass(gather_src):
    mesh = plsc.VectorSubcoreMesh(core_axis_name="c", subcore_axis_name="s",
                                  num_cores=NC, num_subcores=NS)
    scratch = [
        pltpu.VMEM((2, GPC, G), jnp.int32),
        pltpu.VMEM((2, GPC, G), jnp.int32),
        pltpu.VMEM((2, GPC, G, H), jnp.float32),
        pltpu.VMEM_SHARED((N_PAD, H), jnp.float32),
        pltpu.SemaphoreType.DMA,
        pltpu.SemaphoreType.DMA,
    ]

    @functools.partial(
        pl.kernel,
        mesh=mesh,
        out_type=jax.ShapeDtypeStruct((NC, N_PAD, H), jnp.float32),
        scratch_types=scratch,
        compiler_params=pltpu.CompilerParams(use_tc_tiling_on_sc=False),
    )
    def sc_pass(row2d, col2d, src_hbm, zeros_hbm, out_hbm,
                rowv, colv, gath, accum, gsem, ssem):
        cid = lax.axis_index("c")
        sid = lax.axis_index("s")
        wid = sid * NC + cid

        pltpu.sync_copy(zeros_hbm.at[pl.ds(sid * R_TILE, R_TILE)],
                        accum.at[pl.ds(sid * R_TILE, R_TILE)])
        if not gather_src:
            pltpu.sync_copy(src_hbm.at[pl.ds(0, G)], gath.at[0, 0])
        plsc.subcore_barrier()

        gb0 = wid * GPT
        pltpu.sync_copy(row2d.at[pl.ds(gb0, GPC)], rowv.at[0])
        if gather_src:
            pltpu.sync_copy(col2d.at[pl.ds(gb0, GPC)], colv.at[0])
            for j in range(GPC):
                pltpu.async_copy(src_hbm.at[rowv.at[0, j]], gath.at[0, j],
                                 gsem)

        def chunk(c, carry):
            b = lax.rem(c, 2)
            nb = 1 - b

            @pl.when(c > 0)
            def _():
                for j in range(GPC):
                    if gather_src:
                        pltpu.make_async_copy(
                            gath.at[nb, j], accum.at[colv.at[nb, j]],
                            ssem).wait()
                    else:
                        pltpu.make_async_copy(
                            gath.at[0, 0], accum.at[rowv.at[nb, j]],
                            ssem).wait()

            @pl.when(c + 1 < CPT)
            def _():
                gb = wid * GPT + (c + 1) * GPC
                pltpu.sync_copy(row2d.at[pl.ds(gb, GPC)], rowv.at[nb])
                if gather_src:
                    pltpu.sync_copy(col2d.at[pl.ds(gb, GPC)], colv.at[nb])
                    for j in range(GPC):
                        pltpu.async_copy(src_hbm.at[rowv.at[nb, j]],
                                         gath.at[nb, j], gsem)

            if gather_src:
                for j in range(GPC):
                    pltpu.make_async_copy(src_hbm.at[rowv.at[b, j]],
                                          gath.at[b, j], gsem).wait()
                for j in range(GPC):
                    pltpu.async_copy(gath.at[b, j], accum.at[colv.at[b, j]],
                                     ssem, add=True)
            else:
                for j in range(GPC):
                    pltpu.async_copy(gath.at[0, 0], accum.at[rowv.at[b, j]],
                                     ssem, add=True)
            return carry

        lax.fori_loop(0, CPT, chunk, 0)
        lb = (CPT - 1) % 2
        for j in range(GPC):
            if gather_src:
                pltpu.make_async_copy(gath.at[lb, j],
                                      accum.at[colv.at[lb, j]], ssem).wait()
            else:
                pltpu.make_async_copy(gath.at[0, 0],
                                      accum.at[rowv.at[lb, j]], ssem).wait()

        plsc.subcore_barrier()
        pltpu.sync_copy(accum.at[pl.ds(sid * R_TILE, R_TILE)],
                        out_hbm.at[cid, pl.ds(sid * R_TILE, R_TILE)])

    return sc_pass


def _tc1_body(x_ref, w_ref, b_ref, d0_ref, d1_ref, hs_ref, dinv_ref):
    i = pl.program_id(0)
    deg = d0_ref[...] + d1_ref[...] + 1.0
    rid = lax.broadcasted_iota(jnp.int32, (BN, H), 0) + i * BN
    dinv = jnp.where(rid < N, lax.rsqrt(deg), 0.0)
    h = jnp.dot(x_ref[...], w_ref[...], preferred_element_type=jnp.float32)
    hs_ref[...] = dinv * (h + b_ref[...])
    dinv_ref[...] = dinv


def _tc2_body(a0_ref, a1_ref, hs_ref, dinv_ref, w_ref, b_ref, out_ref):
    dinv = dinv_ref[...]
    z = jnp.maximum(dinv * (a0_ref[...] + a1_ref[...] + hs_ref[...]), 0.0)
    h = jnp.dot(z, w_ref[...], preferred_element_type=jnp.float32)
    out_ref[...] = dinv * (h + b_ref[...])


def _tc3_body(a0_ref, a1_ref, hs_ref, dinv_ref, out_ref):
    o = dinv_ref[...] * (a0_ref[...] + a1_ref[...] + hs_ref[...])
    m = jnp.max(o, axis=1, keepdims=True)
    e = jnp.exp(o - m)
    s = jnp.sum(e, axis=1, keepdims=True)
    out_ref[...] = (o - m) - jnp.log(s)


def _row_spec():
    return pl.BlockSpec((BN, H), lambda i: (i, 0))


def _const_spec(shape):
    return pl.BlockSpec(shape, lambda i: (0, 0))


_GRID = N_PAD // BN

_tc1 = pl.pallas_call(
    _tc1_body,
    grid=(_GRID,),
    in_specs=[pl.BlockSpec((BN, D), lambda i: (i, 0)),
              _const_spec((D, H)), _const_spec((1, H)),
              _row_spec(), _row_spec()],
    out_specs=[_row_spec(), _row_spec()],
    out_shape=[jax.ShapeDtypeStruct((N_PAD, H), jnp.float32),
               jax.ShapeDtypeStruct((N_PAD, H), jnp.float32)],
)

_tc2 = pl.pallas_call(
    _tc2_body,
    grid=(_GRID,),
    in_specs=[_row_spec(), _row_spec(), _row_spec(), _row_spec(),
              _const_spec((H, H)), _const_spec((1, H))],
    out_specs=_row_spec(),
    out_shape=jax.ShapeDtypeStruct((N_PAD, H), jnp.float32),
)

_tc3 = pl.pallas_call(
    _tc3_body,
    grid=(_GRID,),
    in_specs=[_row_spec(), _row_spec(), _row_spec(), _row_spec()],
    out_specs=_row_spec(),
    out_shape=jax.ShapeDtypeStruct((N_PAD, H), jnp.float32),
)


def kernel(x, edge_index, W1, b1, W2, b2):
    pad_e = E_PAD - E
    row2d = jnp.concatenate(
        [edge_index[0], jnp.full((pad_e,), N, jnp.int32)]).reshape(E_PAD // G, G)
    col2d = jnp.concatenate(
        [edge_index[1], jnp.full((pad_e,), N, jnp.int32)]).reshape(E_PAD // G, G)

    zeros = jnp.zeros((N_PAD, H), jnp.float32)
    ones_blk = jnp.ones((G, H), jnp.float32)
    x_pad = jnp.concatenate([x, jnp.zeros((N_PAD - N, D), jnp.float32)], axis=0)
    w1t = W1.T
    w2t = W2.T
    b1r = b1.reshape(1, H)
    b2r = b2.reshape(1, H)

    sc_degree = _make_sc_pass(False)
    sc_propagate = _make_sc_pass(True)

    degp = sc_degree(row2d, col2d, ones_blk, zeros)
    hs1, dinv = _tc1(x_pad, w1t, b1r, degp[0], degp[1])
    acc1 = sc_propagate(row2d, col2d, hs1, zeros)
    hs2 = _tc2(acc1[0], acc1[1], hs1, dinv, w2t, b2r)
    acc2 = sc_propagate(row2d, col2d, hs2, zeros)
    out = _tc3(acc2[0], acc2[1], hs2, dinv)
    return out[:N]


# speedup vs baseline: 78.0946x; 1.5578x over previous
import functools

import jax
import jax.numpy as jnp
from jax import lax
from jax.experimental import pallas as pl
from jax.experimental.pallas import tpu as pltpu
import jax.experimental.pallas.tpu_sc as plsc

N = 100000
E = 3200000
D = 128
H = 16

NC = 2
NS = 16
NW = NC * NS
G = 128
GPC = 6
CPT = 131
GPT = GPC * CPT
E_PAD = NW * GPT * G
BN = 2048
N_PAD = 49 * BN
R_TILE = N_PAD // NS

@functools.lru_cache(maxsize=None)
def _make_sc_pass(gather_src):
    mesh = plsc.VectorSubcoreMesh(core_axis_name="c", subcore_axis_name="s",
                                  num_cores=NC, num_subcores=NS)
    scratch = [
        pltpu.VMEM((2, GPC, G), jnp.int32),
        pltpu.VMEM((2, GPC, G), jnp.int32),
        pltpu.VMEM((2, GPC, G, H), jnp.float32),
        pltpu.VMEM_SHARED((N_PAD, H), jnp.float32),
        pltpu.SemaphoreType.DMA,
        pltpu.SemaphoreType.DMA,
    ]

    @functools.partial(
        pl.kernel,
        mesh=mesh,
        out_type=jax.ShapeDtypeStruct((NC, N_PAD, H), jnp.float32),
        scratch_types=scratch,
        compiler_params=pltpu.CompilerParams(use_tc_tiling_on_sc=False),
    )
    def sc_pass(row2d, col2d, src_hbm, zeros_hbm, out_hbm,
                rowv, colv, gath, accum, gsem, ssem):
        cid = lax.axis_index("c")
        sid = lax.axis_index("s")
        wid = sid * NC + cid

        pltpu.sync_copy(zeros_hbm.at[pl.ds(sid * R_TILE, R_TILE)],
                        accum.at[pl.ds(sid * R_TILE, R_TILE)])
        if not gather_src:
            pltpu.sync_copy(src_hbm.at[pl.ds(0, G)], gath.at[0, 0])
        plsc.subcore_barrier()

        gb0 = wid * GPT
        pltpu.sync_copy(row2d.at[pl.ds(gb0, GPC)], rowv.at[0])
        if gather_src:
            pltpu.sync_copy(col2d.at[pl.ds(gb0, GPC)], colv.at[0])
            for j in range(GPC):
                pltpu.async_copy(src_hbm.at[rowv.at[0, j]], gath.at[0, j],
                                 gsem)

        def chunk(c, carry):
            b = lax.rem(c, 2)
            nb = 1 - b

            @pl.when(c > 0)
            def _():
                for j in range(GPC):
                    if gather_src:
                        pltpu.make_async_copy(
                            gath.at[nb, j], accum.at[colv.at[nb, j]],
                            ssem).wait()
                    else:
                        pltpu.make_async_copy(
                            gath.at[0, 0], accum.at[rowv.at[nb, j]],
                            ssem).wait()

            @pl.when(c + 1 < CPT)
            def _():
                gb = wid * GPT + (c + 1) * GPC
                pltpu.sync_copy(row2d.at[pl.ds(gb, GPC)], rowv.at[nb])
                if gather_src:
                    pltpu.sync_copy(col2d.at[pl.ds(gb, GPC)], colv.at[nb])
                    for j in range(GPC):
                        pltpu.async_copy(src_hbm.at[rowv.at[nb, j]],
                                         gath.at[nb, j], gsem)

            if gather_src:
                for j in range(GPC):
                    pltpu.make_async_copy(src_hbm.at[rowv.at[b, j]],
                                          gath.at[b, j], gsem).wait()
                for j in range(GPC):
                    pltpu.async_copy(gath.at[b, j], accum.at[colv.at[b, j]],
                                     ssem, add=True)
            else:
                for j in range(GPC):
                    pltpu.async_copy(gath.at[0, 0], accum.at[rowv.at[b, j]],
                                     ssem, add=True)
            return carry

        lax.fori_loop(0, CPT, chunk, 0)
        lb = (CPT - 1) % 2
        for j in range(GPC):
            if gather_src:
                pltpu.make_async_copy(gath.at[lb, j],
                                      accum.at[colv.at[lb, j]], ssem).wait()
            else:
                pltpu.make_async_copy(gath.at[0, 0],
                                      accum.at[rowv.at[lb, j]], ssem).wait()

        plsc.subcore_barrier()
        pltpu.sync_copy(accum.at[pl.ds(sid * R_TILE, R_TILE)],
                        out_hbm.at[cid, pl.ds(sid * R_TILE, R_TILE)])

    return sc_pass


NP8 = N_PAD // 8
BN8 = 256
_GRID = NP8 // BN8


def _node_dinv(d0, d1, i):
    deg = d0 + d1 + 1.0
    r = lax.broadcasted_iota(jnp.int32, (BN8, 8 * H), 0) + i * BN8
    lane = lax.broadcasted_iota(jnp.int32, (BN8, 8 * H), 1)
    nid = 8 * r + lane // H
    return jnp.where(nid < N, lax.rsqrt(deg), 0.0)


def _tc1_body(x_ref, w_ref, b_ref, d0_ref, d1_ref, hs_ref, dinv_ref):
    i = pl.program_id(0)
    dinv = _node_dinv(d0_ref[0], d1_ref[0], i)
    h = jnp.dot(x_ref[...], w_ref[...], preferred_element_type=jnp.float32)
    hs_ref[...] = dinv * (h + b_ref[...])
    dinv_ref[...] = dinv


def _tc2_body(a0_ref, a1_ref, hs_ref, dinv_ref, w_ref, b_ref, out_ref):
    dinv = dinv_ref[...]
    z = jnp.maximum(dinv * (a0_ref[0] + a1_ref[0] + hs_ref[...]), 0.0)
    h = jnp.dot(z, w_ref[...], preferred_element_type=jnp.float32)
    out_ref[...] = dinv * (h + b_ref[...])


def _tc3_body(a0_ref, a1_ref, hs_ref, dinv_ref, bsum_ref, out_ref):
    o = dinv_ref[...] * (a0_ref[0] + a1_ref[0] + hs_ref[...])
    m = jnp.max(o, axis=1, keepdims=True)
    e = jnp.exp(o - m)
    s = jnp.dot(e, bsum_ref[...], preferred_element_type=jnp.float32)
    out_ref[...] = (o - m) - jnp.log(s)


def _blk():
    return pl.BlockSpec((BN8, 8 * H), lambda i: (i, 0))


def _acc_spec(c):
    return pl.BlockSpec((1, BN8, 8 * H), lambda i, c=c: (c, i, 0))


def _const_spec(shape):
    return pl.BlockSpec(shape, lambda i: tuple(0 for _ in shape))


_tc1 = pl.pallas_call(
    _tc1_body,
    grid=(_GRID,),
    in_specs=[pl.BlockSpec((BN8, 8 * D), lambda i: (i, 0)),
              _const_spec((8 * D, 8 * H)), _const_spec((1, 8 * H)),
              _acc_spec(0), _acc_spec(1)],
    out_specs=[_blk(), _blk()],
    out_shape=[jax.ShapeDtypeStruct((NP8, 8 * H), jnp.float32),
               jax.ShapeDtypeStruct((NP8, 8 * H), jnp.float32)],
)

_tc2 = pl.pallas_call(
    _tc2_body,
    grid=(_GRID,),
    in_specs=[_acc_spec(0), _acc_spec(1), _blk(), _blk(),
              _const_spec((8 * H, 8 * H)), _const_spec((1, 8 * H))],
    out_specs=_blk(),
    out_shape=jax.ShapeDtypeStruct((NP8, 8 * H), jnp.float32),
)

_tc3 = pl.pallas_call(
    _tc3_body,
    grid=(_GRID,),
    in_specs=[_acc_spec(0), _acc_spec(1), _blk(), _blk(),
              _const_spec((8 * H, 8 * H))],
    out_specs=_blk(),
    out_shape=jax.ShapeDtypeStruct((NP8, 8 * H), jnp.float32),
)


def kernel(x, edge_index, W1, b1, W2, b2):
    pad_e = E_PAD - E
    row2d = jnp.concatenate(
        [edge_index[0], jnp.full((pad_e,), N, jnp.int32)]).reshape(E_PAD // G, G)
    col2d = jnp.concatenate(
        [edge_index[1], jnp.full((pad_e,), N, jnp.int32)]).reshape(E_PAD // G, G)

    zeros = jnp.zeros((N_PAD, H), jnp.float32)
    ones_blk = jnp.ones((G, H), jnp.float32)
    x_pad = jnp.concatenate([x, jnp.zeros((N_PAD - N, D), jnp.float32)], axis=0)
    x8 = x_pad.reshape(NP8, 8 * D)
    eye8 = jnp.eye(8, dtype=jnp.float32)
    w1big = jnp.kron(eye8, W1.T)
    w2big = jnp.kron(eye8, W2.T)
    bsum = jnp.kron(eye8, jnp.ones((H, H), jnp.float32))
    b1big = jnp.tile(b1, 8).reshape(1, 8 * H)
    b2big = jnp.tile(b2, 8).reshape(1, 8 * H)

    sc_degree = _make_sc_pass(False)
    sc_propagate = _make_sc_pass(True)

    degp = sc_degree(row2d, col2d, ones_blk, zeros)
    degp8 = degp.reshape(NC, NP8, 8 * H)
    hs1, dinv = _tc1(x8, w1big, b1big, degp8, degp8)
    acc1 = sc_propagate(row2d, col2d, hs1.reshape(N_PAD, H), zeros)
    acc1_8 = acc1.reshape(NC, NP8, 8 * H)
    hs2 = _tc2(acc1_8, acc1_8, hs1, dinv, w2big, b2big)
    acc2 = sc_propagate(row2d, col2d, hs2.reshape(N_PAD, H), zeros)
    acc2_8 = acc2.reshape(NC, NP8, 8 * H)
    out = _tc3(acc2_8, acc2_8, hs2, dinv, bsum)
    return out.reshape(N_PAD, H)[:N]


# speedup vs baseline: 92.8872x; 1.1894x over previous
import functools

import jax
import jax.numpy as jnp
from jax import lax
from jax.experimental import pallas as pl
from jax.experimental.pallas import tpu as pltpu
import jax.experimental.pallas.tpu_sc as plsc

N = 100000
E = 3200000
D = 128
H = 16

NC = 2
NS = 16
NW = NC * NS
G = 128
GPC = 5
CPT = 157
GPT = GPC * CPT
E_PAD = NW * GPT * G
BN = 2048
N_PAD = 49 * BN
R_TILE = N_PAD // NS

@functools.lru_cache(maxsize=None)
def _make_sc_pass(gather_src):
    mesh = plsc.VectorSubcoreMesh(core_axis_name="c", subcore_axis_name="s",
                                  num_cores=NC, num_subcores=NS)
    scratch = [
        pltpu.VMEM((4, GPC, G), jnp.int32),
        pltpu.VMEM((4, GPC, G), jnp.int32),
        pltpu.VMEM((2, GPC, G, H), jnp.float32),
        pltpu.VMEM_SHARED((N_PAD, H), jnp.float32),
        pltpu.SemaphoreType.DMA,
        pltpu.SemaphoreType.DMA,
        pltpu.SemaphoreType.DMA,
    ]

    @functools.partial(
        pl.kernel,
        mesh=mesh,
        out_type=jax.ShapeDtypeStruct((NC, N_PAD, H), jnp.float32),
        scratch_types=scratch,
        compiler_params=pltpu.CompilerParams(use_tc_tiling_on_sc=False),
    )
    def sc_pass(row2d, col2d, src_hbm, zeros_hbm, out_hbm,
                rowv, colv, gath, accum, isem, gsem, ssem):
        cid = lax.axis_index("c")
        sid = lax.axis_index("s")
        wid = sid * NC + cid

        def idx_start(c, slot):
            gb = wid * GPT + c * GPC
            d = [pltpu.async_copy(row2d.at[pl.ds(gb, GPC)], rowv.at[slot],
                                  isem)]
            if gather_src:
                d.append(pltpu.async_copy(col2d.at[pl.ds(gb, GPC)],
                                          colv.at[slot], isem))
            return d

        def idx_wait(slot):
            pltpu.make_async_copy(row2d.at[pl.ds(0, GPC)], rowv.at[slot],
                                  isem).wait()
            if gather_src:
                pltpu.make_async_copy(col2d.at[pl.ds(0, GPC)],
                                      colv.at[slot], isem).wait()

        pltpu.sync_copy(zeros_hbm.at[pl.ds(sid * R_TILE, R_TILE)],
                        accum.at[pl.ds(sid * R_TILE, R_TILE)])
        if not gather_src:
            pltpu.sync_copy(src_hbm.at[pl.ds(0, G)], gath.at[0, 0])
        plsc.subcore_barrier()

        for d in idx_start(0, 0):
            d.wait()
        idx_start(1, 1)
        idx_start(2, 2)
        if gather_src:
            for j in range(GPC):
                pltpu.async_copy(src_hbm.at[rowv.at[0, j]], gath.at[0, j],
                                 gsem)

        def chunk(c, carry):
            b = lax.rem(c, 2)
            nb = 1 - b
            i_cur = lax.rem(c, 4)
            i_nxt = lax.rem(c + 1, 4)
            i_pre = lax.rem(c + 3, 4)

            @pl.when(c > 0)
            def _():
                for j in range(GPC):
                    if gather_src:
                        pltpu.make_async_copy(
                            gath.at[nb, j], accum.at[colv.at[i_pre, j]],
                            ssem).wait()
                    else:
                        pltpu.make_async_copy(
                            gath.at[0, 0], accum.at[rowv.at[i_pre, j]],
                            ssem).wait()

            @pl.when(c + 1 < CPT)
            def _():
                idx_wait(i_nxt)
                if gather_src:
                    for j in range(GPC):
                        pltpu.async_copy(src_hbm.at[rowv.at[i_nxt, j]],
                                         gath.at[nb, j], gsem)

            @pl.when(c + 3 < CPT)
            def _():
                idx_start(c + 3, i_pre)

            for j in range(GPC):
                if gather_src:
                    pltpu.make_async_copy(src_hbm.at[rowv.at[i_cur, j]],
                                          gath.at[b, j], gsem).wait()
                    pltpu.async_copy(gath.at[b, j],
                                     accum.at[colv.at[i_cur, j]],
                                     ssem, add=True)
                else:
                    pltpu.async_copy(gath.at[0, 0],
                                     accum.at[rowv.at[i_cur, j]],
                                     ssem, add=True)
            return carry

        lax.fori_loop(0, CPT, chunk, 0)
        lb = (CPT - 1) % 2
        li = (CPT - 1) % 4
        for j in range(GPC):
            if gather_src:
                pltpu.make_async_copy(gath.at[lb, j],
                                      accum.at[colv.at[li, j]], ssem).wait()
            else:
                pltpu.make_async_copy(gath.at[0, 0],
                                      accum.at[rowv.at[li, j]], ssem).wait()

        plsc.subcore_barrier()
        pltpu.sync_copy(accum.at[pl.ds(sid * R_TILE, R_TILE)],
                        out_hbm.at[cid, pl.ds(sid * R_TILE, R_TILE)])

    return sc_pass


NP8 = N_PAD // 8
BN8 = 256
_GRID = NP8 // BN8


def _node_dinv(d0, d1, i):
    deg = d0 + d1 + 1.0
    r = lax.broadcasted_iota(jnp.int32, (BN8, 8 * H), 0) + i * BN8
    lane = lax.broadcasted_iota(jnp.int32, (BN8, 8 * H), 1)
    nid = 8 * r + lane // H
    return jnp.where(nid < N, lax.rsqrt(deg), 0.0)


def _tc1_body(x_ref, w_ref, b_ref, d0_ref, d1_ref, hs_ref, dinv_ref):
    i = pl.program_id(0)
    dinv = _node_dinv(d0_ref[0], d1_ref[0], i)
    h = jnp.dot(x_ref[...], w_ref[...], preferred_element_type=jnp.float32)
    hs_ref[...] = dinv * (h + b_ref[...])
    dinv_ref[...] = dinv


def _tc2_body(a0_ref, a1_ref, hs_ref, dinv_ref, w_ref, b_ref, out_ref):
    dinv = dinv_ref[...]
    z = jnp.maximum(dinv * (a0_ref[0] + a1_ref[0] + hs_ref[...]), 0.0)
    h = jnp.dot(z, w_ref[...], preferred_element_type=jnp.float32)
    out_ref[...] = dinv * (h + b_ref[...])


def _tc3_body(a0_ref, a1_ref, hs_ref, dinv_ref, bsum_ref, out_ref):
    o = dinv_ref[...] * (a0_ref[0] + a1_ref[0] + hs_ref[...])
    m = jnp.max(o, axis=1, keepdims=True)
    e = jnp.exp(o - m)
    s = jnp.dot(e, bsum_ref[...], preferred_element_type=jnp.float32)
    out_ref[...] = (o - m) - jnp.log(s)


def _blk():
    return pl.BlockSpec((BN8, 8 * H), lambda i: (i, 0))


def _acc_spec(c):
    return pl.BlockSpec((1, BN8, 8 * H), lambda i, c=c: (c, i, 0))


def _const_spec(shape):
    return pl.BlockSpec(shape, lambda i: tuple(0 for _ in shape))


_tc1 = pl.pallas_call(
    _tc1_body,
    grid=(_GRID,),
    in_specs=[pl.BlockSpec((BN8, 8 * D), lambda i: (i, 0)),
              _const_spec((8 * D, 8 * H)), _const_spec((1, 8 * H)),
              _acc_spec(0), _acc_spec(1)],
    out_specs=[_blk(), _blk()],
    out_shape=[jax.ShapeDtypeStruct((NP8, 8 * H), jnp.float32),
               jax.ShapeDtypeStruct((NP8, 8 * H), jnp.float32)],
)

_tc2 = pl.pallas_call(
    _tc2_body,
    grid=(_GRID,),
    in_specs=[_acc_spec(0), _acc_spec(1), _blk(), _blk(),
              _const_spec((8 * H, 8 * H)), _const_spec((1, 8 * H))],
    out_specs=_blk(),
    out_shape=jax.ShapeDtypeStruct((NP8, 8 * H), jnp.float32),
)

_tc3 = pl.pallas_call(
    _tc3_body,
    grid=(_GRID,),
    in_specs=[_acc_spec(0), _acc_spec(1), _blk(), _blk(),
              _const_spec((8 * H, 8 * H))],
    out_specs=_blk(),
    out_shape=jax.ShapeDtypeStruct((NP8, 8 * H), jnp.float32),
)


def kernel(x, edge_index, W1, b1, W2, b2):
    pad_e = E_PAD - E
    row2d = jnp.concatenate(
        [edge_index[0], jnp.full((pad_e,), N, jnp.int32)]).reshape(E_PAD // G, G)
    col2d = jnp.concatenate(
        [edge_index[1], jnp.full((pad_e,), N, jnp.int32)]).reshape(E_PAD // G, G)

    zeros = jnp.zeros((N_PAD, H), jnp.float32)
    ones_blk = jnp.ones((G, H), jnp.float32)
    x_pad = jnp.concatenate([x, jnp.zeros((N_PAD - N, D), jnp.float32)], axis=0)
    x8 = x_pad.reshape(NP8, 8 * D)
    eye8 = jnp.eye(8, dtype=jnp.float32)
    w1big = jnp.kron(eye8, W1.T)
    w2big = jnp.kron(eye8, W2.T)
    bsum = jnp.kron(eye8, jnp.ones((H, H), jnp.float32))
    b1big = jnp.tile(b1, 8).reshape(1, 8 * H)
    b2big = jnp.tile(b2, 8).reshape(1, 8 * H)

    sc_degree = _make_sc_pass(False)
    sc_propagate = _make_sc_pass(True)

    degp = sc_degree(row2d, col2d, ones_blk, zeros)
    degp8 = degp.reshape(NC, NP8, 8 * H)
    hs1, dinv = _tc1(x8, w1big, b1big, degp8, degp8)
    acc1 = sc_propagate(row2d, col2d, hs1.reshape(N_PAD, H), zeros)
    acc1_8 = acc1.reshape(NC, NP8, 8 * H)
    hs2 = _tc2(acc1_8, acc1_8, hs1, dinv, w2big, b2big)
    acc2 = sc_propagate(row2d, col2d, hs2.reshape(N_PAD, H), zeros)
    acc2_8 = acc2.reshape(NC, NP8, 8 * H)
    out = _tc3(acc2_8, acc2_8, hs2, dinv, bsum)
    return out.reshape(N_PAD, H)[:N]


# speedup vs baseline: 94.5144x; 1.0175x over previous
import functools

import jax
import jax.numpy as jnp
from jax import lax
from jax.experimental import pallas as pl
from jax.experimental.pallas import tpu as pltpu
import jax.experimental.pallas.tpu_sc as plsc

N = 100000
E = 3200000
D = 128
H = 16

NC = 2
NS = 16
NW = NC * NS
G = 128
GPC = 5
CPT = 157
GPT = GPC * CPT
E_PAD = NW * GPT * G
BN = 2048
N_PAD = 49 * BN
R_TILE = N_PAD // NS

@functools.lru_cache(maxsize=None)
def _make_sc_pass(gather_src):
    mesh = plsc.VectorSubcoreMesh(core_axis_name="c", subcore_axis_name="s",
                                  num_cores=NC, num_subcores=NS)
    scratch = [
        pltpu.VMEM((4, GPC, G), jnp.int32),
        pltpu.VMEM((4, GPC, G), jnp.int32),
        pltpu.VMEM((2, GPC, G, H), jnp.float32),
        pltpu.VMEM_SHARED((N_PAD, H), jnp.float32),
        pltpu.SemaphoreType.DMA,
        pltpu.SemaphoreType.DMA,
        pltpu.SemaphoreType.DMA,
    ]

    @functools.partial(
        pl.kernel,
        mesh=mesh,
        out_type=jax.ShapeDtypeStruct((NC, N_PAD, H), jnp.float32),
        scratch_types=scratch,
        compiler_params=pltpu.CompilerParams(use_tc_tiling_on_sc=False),
    )
    def sc_pass(row2d, col2d, src_hbm, zeros_hbm, out_hbm,
                rowv, colv, gath, accum, isem, gsem, ssem):
        cid = lax.axis_index("c")
        sid = lax.axis_index("s")
        wid = sid * NC + cid

        def idx_start(c, slot):
            gb = wid * GPT + c * GPC
            d = [pltpu.async_copy(row2d.at[pl.ds(gb, GPC)], rowv.at[slot],
                                  isem)]
            if gather_src:
                d.append(pltpu.async_copy(col2d.at[pl.ds(gb, GPC)],
                                          colv.at[slot], isem))
            return d

        def idx_wait(slot):
            pltpu.make_async_copy(row2d.at[pl.ds(0, GPC)], rowv.at[slot],
                                  isem).wait()
            if gather_src:
                pltpu.make_async_copy(col2d.at[pl.ds(0, GPC)],
                                      colv.at[slot], isem).wait()

        pltpu.sync_copy(zeros_hbm.at[pl.ds(sid * R_TILE, R_TILE)],
                        accum.at[pl.ds(sid * R_TILE, R_TILE)])
        if not gather_src:
            pltpu.sync_copy(src_hbm.at[pl.ds(0, G)], gath.at[0, 0])
        plsc.subcore_barrier()

        for d in idx_start(0, 0):
            d.wait()
        idx_start(1, 1)
        idx_start(2, 2)
        if gather_src:
            for j in range(GPC):
                pltpu.async_copy(src_hbm.at[rowv.at[0, j]], gath.at[0, j],
                                 gsem)

        def chunk(c, carry):
            b = lax.rem(c, 2)
            nb = 1 - b
            i_cur = lax.rem(c, 4)
            i_nxt = lax.rem(c + 1, 4)
            i_pre = lax.rem(c + 3, 4)

            @pl.when(c > 0)
            def _():
                for j in range(GPC):
                    if gather_src:
                        pltpu.make_async_copy(
                            gath.at[nb, j], accum.at[colv.at[i_pre, j]],
                            ssem).wait()
                    else:
                        pltpu.make_async_copy(
                            gath.at[0, 0], accum.at[rowv.at[i_pre, j]],
                            ssem).wait()

            @pl.when(c + 1 < CPT)
            def _():
                idx_wait(i_nxt)
                if gather_src:
                    for j in range(GPC):
                        pltpu.async_copy(src_hbm.at[rowv.at[i_nxt, j]],
                                         gath.at[nb, j], gsem)

            @pl.when(c + 3 < CPT)
            def _():
                idx_start(c + 3, i_pre)

            for j in range(GPC):
                if gather_src:
                    pltpu.make_async_copy(src_hbm.at[rowv.at[i_cur, j]],
                                          gath.at[b, j], gsem).wait()
                    pltpu.async_copy(gath.at[b, j],
                                     accum.at[colv.at[i_cur, j]],
                                     ssem, add=True)
                else:
                    pltpu.async_copy(gath.at[0, 0],
                                     accum.at[rowv.at[i_cur, j]],
                                     ssem, add=True)
            return carry

        lax.fori_loop(0, CPT, chunk, 0)
        lb = (CPT - 1) % 2
        li = (CPT - 1) % 4
        for j in range(GPC):
            if gather_src:
                pltpu.make_async_copy(gath.at[lb, j],
                                      accum.at[colv.at[li, j]], ssem).wait()
            else:
                pltpu.make_async_copy(gath.at[0, 0],
                                      accum.at[rowv.at[li, j]], ssem).wait()

        plsc.subcore_barrier()
        pltpu.sync_copy(accum.at[pl.ds(sid * R_TILE, R_TILE)],
                        out_hbm.at[cid, pl.ds(sid * R_TILE, R_TILE)])

    return sc_pass


NP8 = N_PAD // 8
BN8 = 256
_GRID = NP8 // BN8


def _node_dinv(d0, d1, i):
    deg = d0 + d1 + 1.0
    r = lax.broadcasted_iota(jnp.int32, (BN8, 8 * H), 0) + i * BN8
    lane = lax.broadcasted_iota(jnp.int32, (BN8, 8 * H), 1)
    nid = 8 * r + lane // H
    return jnp.where(nid < N, lax.rsqrt(deg), 0.0)


def _tc1_body(x_ref, w_ref, b_ref, d0_ref, d1_ref, hs_ref, dinv_ref):
    i = pl.program_id(0)
    dinv = _node_dinv(d0_ref[0], d1_ref[0], i)
    h = jnp.dot(x_ref[...], w_ref[...], preferred_element_type=jnp.float32)
    hs_ref[...] = dinv * (h + b_ref[...])
    dinv_ref[...] = dinv


def _tc2_body(a0_ref, a1_ref, hs_ref, dinv_ref, w_ref, b_ref, out_ref):
    dinv = dinv_ref[...]
    z = jnp.maximum(dinv * (a0_ref[0] + a1_ref[0] + hs_ref[...]), 0.0)
    h = jnp.dot(z, w_ref[...], preferred_element_type=jnp.float32)
    out_ref[...] = dinv * (h + b_ref[...])


def _tc3_body(a0_ref, a1_ref, hs_ref, dinv_ref, bsum_ref, out_ref):
    o = dinv_ref[...] * (a0_ref[0] + a1_ref[0] + hs_ref[...])
    m = jnp.max(o, axis=1, keepdims=True)
    e = jnp.exp(o - m)
    s = jnp.dot(e, bsum_ref[...], preferred_element_type=jnp.float32)
    out_ref[...] = (o - m) - jnp.log(s)


def _blk():
    return pl.BlockSpec((BN8, 8 * H), lambda i: (i, 0))


def _acc_spec(c):
    return pl.BlockSpec((1, BN8, 8 * H), lambda i, c=c: (c, i, 0))


def _const_spec(shape):
    return pl.BlockSpec(shape, lambda i: tuple(0 for _ in shape))


_tc1 = pl.pallas_call(
    _tc1_body,
    grid=(_GRID,),
    in_specs=[pl.BlockSpec((BN8, 8 * D), lambda i: (i, 0)),
              _const_spec((8 * D, 8 * H)), _const_spec((1, 8 * H)),
              _acc_spec(0), _acc_spec(1)],
    out_specs=[_blk(), _blk()],
    out_shape=[jax.ShapeDtypeStruct((NP8, 8 * H), jnp.float32),
               jax.ShapeDtypeStruct((NP8, 8 * H), jnp.float32)],
)

_tc2 = pl.pallas_call(
    _tc2_body,
    grid=(_GRID,),
    in_specs=[_acc_spec(0), _acc_spec(1), _blk(), _blk(),
              _const_spec((8 * H, 8 * H)), _const_spec((1, 8 * H))],
    out_specs=_blk(),
    out_shape=jax.ShapeDtypeStruct((NP8, 8 * H), jnp.float32),
)

_tc3 = pl.pallas_call(
    _tc3_body,
    grid=(_GRID,),
    in_specs=[_acc_spec(0), _acc_spec(1), _blk(), _blk(),
              _const_spec((8 * H, 8 * H))],
    out_specs=_blk(),
    out_shape=jax.ShapeDtypeStruct((NP8, 8 * H), jnp.float32),
)


def kernel(x, edge_index, W1, b1, W2, b2):
    pad_e = E_PAD - E
    row2d = jnp.concatenate(
        [edge_index[0], jnp.full((pad_e,), N, jnp.int32)]).reshape(E_PAD // G, G)
    col2d = jnp.concatenate(
        [edge_index[1], jnp.full((pad_e,), N, jnp.int32)]).reshape(E_PAD // G, G)

    zeros = jnp.zeros((N_PAD, H), jnp.float32)
    ones_blk = jnp.ones((G, H), jnp.float32)
    x8 = x.reshape(N // 8, 8 * D)
    eye8 = jnp.eye(8, dtype=jnp.float32)
    w1big = jnp.kron(eye8, W1.T)
    w2big = jnp.kron(eye8, W2.T)
    bsum = jnp.kron(eye8, jnp.ones((H, H), jnp.float32))
    b1big = jnp.tile(b1, 8).reshape(1, 8 * H)
    b2big = jnp.tile(b2, 8).reshape(1, 8 * H)

    sc_degree = _make_sc_pass(False)
    sc_propagate = _make_sc_pass(True)

    degp = sc_degree(row2d, col2d, ones_blk, zeros)
    degp8 = degp.reshape(NC, NP8, 8 * H)
    hs1, dinv = _tc1(x8, w1big, b1big, degp8, degp8)
    acc1 = sc_propagate(row2d, col2d, hs1.reshape(N_PAD, H), zeros)
    acc1_8 = acc1.reshape(NC, NP8, 8 * H)
    hs2 = _tc2(acc1_8, acc1_8, hs1, dinv, w2big, b2big)
    acc2 = sc_propagate(row2d, col2d, hs2.reshape(N_PAD, H), zeros)
    acc2_8 = acc2.reshape(NC, NP8, 8 * H)
    out = _tc3(acc2_8, acc2_8, hs2, dinv, bsum)
    return out.reshape(N_PAD, H)[:N]


# speedup vs baseline: 106.1963x; 1.1236x over previous
import functools

import jax
import jax.numpy as jnp
from jax import lax
from jax.experimental import pallas as pl
from jax.experimental.pallas import tpu as pltpu
import jax.experimental.pallas.tpu_sc as plsc

N = 100000
E = 3200000
D = 128
H = 16

NC = 2
NS = 16
NW = NC * NS
G = 128
GPC = 4
CPT = 196
GPT = GPC * CPT
E_PAD = NW * GPT * G
BN = 2048
N_PAD = 49 * BN
R_TILE = N_PAD // NS

@functools.lru_cache(maxsize=None)
def _make_sc_pass(gather_src):
    mesh = plsc.VectorSubcoreMesh(core_axis_name="c", subcore_axis_name="s",
                                  num_cores=NC, num_subcores=NS)
    scratch = [
        pltpu.VMEM((5, GPC, G), jnp.int32),
        pltpu.VMEM((5, GPC, G), jnp.int32),
        pltpu.VMEM((3, GPC, G, H), jnp.float32),
        pltpu.VMEM_SHARED((N_PAD, H), jnp.float32),
        pltpu.SemaphoreType.DMA,
        pltpu.SemaphoreType.DMA,
        pltpu.SemaphoreType.DMA,
    ]

    @functools.partial(
        pl.kernel,
        mesh=mesh,
        out_type=jax.ShapeDtypeStruct((NC, N_PAD, H), jnp.float32),
        scratch_types=scratch,
        compiler_params=pltpu.CompilerParams(use_tc_tiling_on_sc=False),
    )
    def sc_pass(row2d, col2d, src_hbm, zeros_hbm, out_hbm,
                rowv, colv, gath, accum, isem, gsem, ssem):
        cid = lax.axis_index("c")
        sid = lax.axis_index("s")
        wid = sid * NC + cid

        def idx_start(c, slot):
            gb = wid * GPT + c * GPC
            d = [pltpu.async_copy(row2d.at[pl.ds(gb, GPC)], rowv.at[slot],
                                  isem)]
            if gather_src:
                d.append(pltpu.async_copy(col2d.at[pl.ds(gb, GPC)],
                                          colv.at[slot], isem))
            return d

        def idx_wait(slot):
            pltpu.make_async_copy(row2d.at[pl.ds(0, GPC)], rowv.at[slot],
                                  isem).wait()
            if gather_src:
                pltpu.make_async_copy(col2d.at[pl.ds(0, GPC)],
                                      colv.at[slot], isem).wait()

        pltpu.sync_copy(zeros_hbm.at[pl.ds(sid * R_TILE, R_TILE)],
                        accum.at[pl.ds(sid * R_TILE, R_TILE)])
        if not gather_src:
            pltpu.sync_copy(src_hbm.at[pl.ds(0, G)], gath.at[0, 0])
        plsc.subcore_barrier()

        for d in idx_start(0, 0):
            d.wait()
        for d in idx_start(1, 1):
            d.wait()
        idx_start(2, 2)
        idx_start(3, 3)
        if gather_src:
            for s in (0, 1):
                for j in range(GPC):
                    pltpu.async_copy(src_hbm.at[rowv.at[s, j]],
                                     gath.at[s, j], gsem)

        def chunk(c, carry):
            g_cur = lax.rem(c, 3)
            g_nxt = lax.rem(c + 2, 3)
            g_prv = lax.rem(c + 2, 3)
            i_cur = lax.rem(c, 5)
            i_nxt = lax.rem(c + 2, 5)
            i_prv = lax.rem(c + 4, 5)

            @pl.when(c > 0)
            def _():
                for j in range(GPC):
                    if gather_src:
                        pltpu.make_async_copy(
                            gath.at[g_prv, j], accum.at[colv.at[i_prv, j]],
                            ssem).wait()
                    else:
                        pltpu.make_async_copy(
                            gath.at[0, 0], accum.at[rowv.at[i_prv, j]],
                            ssem).wait()

            @pl.when(c + 2 < CPT)
            def _():
                idx_wait(i_nxt)
                if gather_src:
                    for j in range(GPC):
                        pltpu.async_copy(src_hbm.at[rowv.at[i_nxt, j]],
                                         gath.at[g_nxt, j], gsem)

            @pl.when(c + 4 < CPT)
            def _():
                idx_start(c + 4, i_prv)

            for j in range(GPC):
                if gather_src:
                    pltpu.make_async_copy(src_hbm.at[rowv.at[i_cur, j]],
                                          gath.at[g_cur, j], gsem).wait()
                    pltpu.async_copy(gath.at[g_cur, j],
                                     accum.at[colv.at[i_cur, j]],
                                     ssem, add=True)
                else:
                    pltpu.async_copy(gath.at[0, 0],
                                     accum.at[rowv.at[i_cur, j]],
                                     ssem, add=True)
            return carry

        lax.fori_loop(0, CPT, chunk, 0)
        lb = (CPT - 1) % 3
        li = (CPT - 1) % 5
        for j in range(GPC):
            if gather_src:
                pltpu.make_async_copy(gath.at[lb, j],
                                      accum.at[colv.at[li, j]], ssem).wait()
            else:
                pltpu.make_async_copy(gath.at[0, 0],
                                      accum.at[rowv.at[li, j]], ssem).wait()

        plsc.subcore_barrier()
        pltpu.sync_copy(accum.at[pl.ds(sid * R_TILE, R_TILE)],
                        out_hbm.at[cid, pl.ds(sid * R_TILE, R_TILE)])

    return sc_pass


NP8 = N_PAD // 8
BN8 = 256
_GRID = NP8 // BN8


def _node_dinv(d0, d1, i):
    deg = d0 + d1 + 1.0
    r = lax.broadcasted_iota(jnp.int32, (BN8, 8 * H), 0) + i * BN8
    lane = lax.broadcasted_iota(jnp.int32, (BN8, 8 * H), 1)
    nid = 8 * r + lane // H
    return jnp.where(nid < N, lax.rsqrt(deg), 0.0)


def _tc1_body(x_ref, w_ref, b_ref, d0_ref, d1_ref, hs_ref, dinv_ref):
    i = pl.program_id(0)
    dinv = _node_dinv(d0_ref[0], d1_ref[0], i)
    h = jnp.dot(x_ref[...], w_ref[...], preferred_element_type=jnp.float32)
    hs_ref[...] = dinv * (h + b_ref[...])
    dinv_ref[...] = dinv


def _tc2_body(a0_ref, a1_ref, hs_ref, dinv_ref, w_ref, b_ref, out_ref):
    dinv = dinv_ref[...]
    z = jnp.maximum(dinv * (a0_ref[0] + a1_ref[0] + hs_ref[...]), 0.0)
    h = jnp.dot(z, w_ref[...], preferred_element_type=jnp.float32)
    out_ref[...] = dinv * (h + b_ref[...])


def _tc3_body(a0_ref, a1_ref, hs_ref, dinv_ref, bsum_ref, out_ref):
    o = dinv_ref[...] * (a0_ref[0] + a1_ref[0] + hs_ref[...])
    m = jnp.max(o, axis=1, keepdims=True)
    e = jnp.exp(o - m)
    s = jnp.dot(e, bsum_ref[...], preferred_element_type=jnp.float32)
    out_ref[...] = (o - m) - jnp.log(s)


def _blk():
    return pl.BlockSpec((BN8, 8 * H), lambda i: (i, 0))


def _acc_spec(c):
    return pl.BlockSpec((1, BN8, 8 * H), lambda i, c=c: (c, i, 0))


def _const_spec(shape):
    return pl.BlockSpec(shape, lambda i: tuple(0 for _ in shape))


_tc1 = pl.pallas_call(
    _tc1_body,
    grid=(_GRID,),
    in_specs=[pl.BlockSpec((BN8, 8 * D), lambda i: (i, 0)),
              _const_spec((8 * D, 8 * H)), _const_spec((1, 8 * H)),
              _acc_spec(0), _acc_spec(1)],
    out_specs=[_blk(), _blk()],
    out_shape=[jax.ShapeDtypeStruct((NP8, 8 * H), jnp.float32),
               jax.ShapeDtypeStruct((NP8, 8 * H), jnp.float32)],
)

_tc2 = pl.pallas_call(
    _tc2_body,
    grid=(_GRID,),
    in_specs=[_acc_spec(0), _acc_spec(1), _blk(), _blk(),
              _const_spec((8 * H, 8 * H)), _const_spec((1, 8 * H))],
    out_specs=_blk(),
    out_shape=jax.ShapeDtypeStruct((NP8, 8 * H), jnp.float32),
)

_tc3 = pl.pallas_call(
    _tc3_body,
    grid=(_GRID,),
    in_specs=[_acc_spec(0), _acc_spec(1), _blk(), _blk(),
              _const_spec((8 * H, 8 * H))],
    out_specs=_blk(),
    out_shape=jax.ShapeDtypeStruct((NP8, 8 * H), jnp.float32),
)


def kernel(x, edge_index, W1, b1, W2, b2):
    pad_e = E_PAD - E
    row2d = jnp.concatenate(
        [edge_index[0], jnp.full((pad_e,), N, jnp.int32)]).reshape(E_PAD // G, G)
    col2d = jnp.concatenate(
        [edge_index[1], jnp.full((pad_e,), N, jnp.int32)]).reshape(E_PAD // G, G)

    zeros = jnp.zeros((N_PAD, H), jnp.float32)
    ones_blk = jnp.ones((G, H), jnp.float32)
    x8 = x.reshape(N // 8, 8 * D)
    eye8 = jnp.eye(8, dtype=jnp.float32)
    w1big = jnp.kron(eye8, W1.T)
    w2big = jnp.kron(eye8, W2.T)
    bsum = jnp.kron(eye8, jnp.ones((H, H), jnp.float32))
    b1big = jnp.tile(b1, 8).reshape(1, 8 * H)
    b2big = jnp.tile(b2, 8).reshape(1, 8 * H)

    sc_degree = _make_sc_pass(False)
    sc_propagate = _make_sc_pass(True)

    degp = sc_degree(row2d, col2d, ones_blk, zeros)
    degp8 = degp.reshape(NC, NP8, 8 * H)
    hs1, dinv = _tc1(x8, w1big, b1big, degp8, degp8)
    acc1 = sc_propagate(row2d, col2d, hs1.reshape(N_PAD, H), zeros)
    acc1_8 = acc1.reshape(NC, NP8, 8 * H)
    hs2 = _tc2(acc1_8, acc1_8, hs1, dinv, w2big, b2big)
    acc2 = sc_propagate(row2d, col2d, hs2.reshape(N_PAD, H), zeros)
    acc2_8 = acc2.reshape(NC, NP8, 8 * H)
    out = _tc3(acc2_8, acc2_8, hs2, dinv, bsum)
    return out.reshape(N_PAD, H)[:N]


# speedup vs baseline: 110.2767x; 1.0384x over previous
import functools

import jax
import jax.numpy as jnp
from jax import lax
from jax.experimental import pallas as pl
from jax.experimental.pallas import tpu as pltpu
import jax.experimental.pallas.tpu_sc as plsc

N = 100000
E = 3200000
D = 128
H = 16

NC = 2
NS = 16
NW = NC * NS
G = 128
GPC = 4
CPT = 196
GPT = GPC * CPT
E_PAD = NW * GPT * G
BN = 2048
N_PAD = 49 * BN
R_TILE = N_PAD // NS

@functools.lru_cache(maxsize=None)
def _make_sc_pass(gather_src):
    mesh = plsc.VectorSubcoreMesh(core_axis_name="c", subcore_axis_name="s",
                                  num_cores=NC, num_subcores=NS)
    scratch = [
        pltpu.VMEM((5, GPC, G), jnp.int32),
        pltpu.VMEM((5, GPC, G), jnp.int32),
        pltpu.VMEM((3, GPC, G, H), jnp.float32),
        pltpu.VMEM_SHARED((N_PAD, H), jnp.float32),
        pltpu.SemaphoreType.DMA((5,)),
        pltpu.SemaphoreType.DMA((3,)),
        pltpu.SemaphoreType.DMA,
    ]

    @functools.partial(
        pl.kernel,
        mesh=mesh,
        out_type=jax.ShapeDtypeStruct((NC, N_PAD, H), jnp.float32),
        scratch_types=scratch,
        compiler_params=pltpu.CompilerParams(use_tc_tiling_on_sc=False),
    )
    def sc_pass(row2d, col2d, src_hbm, zeros_hbm, out_hbm,
                rowv, colv, gath, accum, isem, gsem, ssem):
        cid = lax.axis_index("c")
        sid = lax.axis_index("s")
        wid = sid * NC + cid

        def idx_start(c, slot):
            gb = wid * GPT + c * GPC
            d = [pltpu.async_copy(row2d.at[pl.ds(gb, GPC)], rowv.at[slot],
                                  isem.at[slot])]
            if gather_src:
                d.append(pltpu.async_copy(col2d.at[pl.ds(gb, GPC)],
                                          colv.at[slot], isem.at[slot]))
            return d

        def idx_wait(slot):
            pltpu.make_async_copy(row2d.at[pl.ds(0, GPC)], rowv.at[slot],
                                  isem.at[slot]).wait()
            if gather_src:
                pltpu.make_async_copy(col2d.at[pl.ds(0, GPC)],
                                      colv.at[slot], isem.at[slot]).wait()

        pltpu.sync_copy(zeros_hbm.at[pl.ds(sid * R_TILE, R_TILE)],
                        accum.at[pl.ds(sid * R_TILE, R_TILE)])
        if not gather_src:
            pltpu.sync_copy(src_hbm.at[pl.ds(0, G)], gath.at[0, 0])
        plsc.subcore_barrier()

        for d in idx_start(0, 0):
            d.wait()
        for d in idx_start(1, 1):
            d.wait()
        idx_start(2, 2)
        idx_start(3, 3)
        if gather_src:
            for s in (0, 1):
                for j in range(GPC):
                    pltpu.async_copy(src_hbm.at[rowv.at[s, j]],
                                     gath.at[s, j], gsem.at[s])

        def chunk(c, carry):
            g_cur = lax.rem(c, 3)
            g_nxt = lax.rem(c + 2, 3)
            g_prv = lax.rem(c + 2, 3)
            i_cur = lax.rem(c, 5)
            i_nxt = lax.rem(c + 2, 5)
            i_prv = lax.rem(c + 4, 5)

            @pl.when(c > 0)
            def _():
                for j in range(GPC):
                    if gather_src:
                        pltpu.make_async_copy(
                            gath.at[g_prv, j], accum.at[colv.at[i_prv, j]],
                            ssem).wait()
                    else:
                        pltpu.make_async_copy(
                            gath.at[0, 0], accum.at[rowv.at[i_prv, j]],
                            ssem).wait()

            @pl.when(c + 2 < CPT)
            def _():
                idx_wait(i_nxt)
                if gather_src:
                    for j in range(GPC):
                        pltpu.async_copy(src_hbm.at[rowv.at[i_nxt, j]],
                                         gath.at[g_nxt, j], gsem.at[g_nxt])

            @pl.when(c + 4 < CPT)
            def _():
                idx_start(c + 4, i_prv)

            for j in range(GPC):
                if gather_src:
                    pltpu.make_async_copy(src_hbm.at[rowv.at[i_cur, j]],
                                          gath.at[g_cur, j],
                                          gsem.at[g_cur]).wait()
                    pltpu.async_copy(gath.at[g_cur, j],
                                     accum.at[colv.at[i_cur, j]],
                                     ssem, add=True)
                else:
                    pltpu.async_copy(gath.at[0, 0],
                                     accum.at[rowv.at[i_cur, j]],
                                     ssem, add=True)
            return carry

        lax.fori_loop(0, CPT, chunk, 0)
        lb = (CPT - 1) % 3
        li = (CPT - 1) % 5
        for j in range(GPC):
            if gather_src:
                pltpu.make_async_copy(gath.at[lb, j],
                                      accum.at[colv.at[li, j]], ssem).wait()
            else:
                pltpu.make_async_copy(gath.at[0, 0],
                                      accum.at[rowv.at[li, j]], ssem).wait()

        plsc.subcore_barrier()
        pltpu.sync_copy(accum.at[pl.ds(sid * R_TILE, R_TILE)],
                        out_hbm.at[cid, pl.ds(sid * R_TILE, R_TILE)])

    return sc_pass


NP8 = N_PAD // 8
BN8 = 256
_GRID = NP8 // BN8


def _node_dinv(d0, d1, i):
    deg = d0 + d1 + 1.0
    r = lax.broadcasted_iota(jnp.int32, (BN8, 8 * H), 0) + i * BN8
    lane = lax.broadcasted_iota(jnp.int32, (BN8, 8 * H), 1)
    nid = 8 * r + lane // H
    return jnp.where(nid < N, lax.rsqrt(deg), 0.0)


def _tc1_body(x_ref, w_ref, b_ref, d0_ref, d1_ref, hs_ref, dinv_ref):
    i = pl.program_id(0)
    dinv = _node_dinv(d0_ref[0], d1_ref[0], i)
    h = jnp.dot(x_ref[...], w_ref[...], preferred_element_type=jnp.float32)
    hs_ref[...] = dinv * (h + b_ref[...])
    dinv_ref[...] = dinv


def _tc2_body(a0_ref, a1_ref, hs_ref, dinv_ref, w_ref, b_ref, out_ref):
    dinv = dinv_ref[...]
    z = jnp.maximum(dinv * (a0_ref[0] + a1_ref[0] + hs_ref[...]), 0.0)
    h = jnp.dot(z, w_ref[...], preferred_element_type=jnp.float32)
    out_ref[...] = dinv * (h + b_ref[...])


def _tc3_body(a0_ref, a1_ref, hs_ref, dinv_ref, bsum_ref, out_ref):
    o = dinv_ref[...] * (a0_ref[0] + a1_ref[0] + hs_ref[...])
    m = jnp.max(o, axis=1, keepdims=True)
    e = jnp.exp(o - m)
    s = jnp.dot(e, bsum_ref[...], preferred_element_type=jnp.float32)
    out_ref[...] = (o - m) - jnp.log(s)


def _blk():
    return pl.BlockSpec((BN8, 8 * H), lambda i: (i, 0))


def _acc_spec(c):
    return pl.BlockSpec((1, BN8, 8 * H), lambda i, c=c: (c, i, 0))


def _const_spec(shape):
    return pl.BlockSpec(shape, lambda i: tuple(0 for _ in shape))


_tc1 = pl.pallas_call(
    _tc1_body,
    grid=(_GRID,),
    in_specs=[pl.BlockSpec((BN8, 8 * D), lambda i: (i, 0)),
              _const_spec((8 * D, 8 * H)), _const_spec((1, 8 * H)),
              _acc_spec(0), _acc_spec(1)],
    out_specs=[_blk(), _blk()],
    out_shape=[jax.ShapeDtypeStruct((NP8, 8 * H), jnp.float32),
               jax.ShapeDtypeStruct((NP8, 8 * H), jnp.float32)],
)

_tc2 = pl.pallas_call(
    _tc2_body,
    grid=(_GRID,),
    in_specs=[_acc_spec(0), _acc_spec(1), _blk(), _blk(),
              _const_spec((8 * H, 8 * H)), _const_spec((1, 8 * H))],
    out_specs=_blk(),
    out_shape=jax.ShapeDtypeStruct((NP8, 8 * H), jnp.float32),
)

_tc3 = pl.pallas_call(
    _tc3_body,
    grid=(_GRID,),
    in_specs=[_acc_spec(0), _acc_spec(1), _blk(), _blk(),
              _const_spec((8 * H, 8 * H))],
    out_specs=_blk(),
    out_shape=jax.ShapeDtypeStruct((NP8, 8 * H), jnp.float32),
)


def kernel(x, edge_index, W1, b1, W2, b2):
    pad_e = E_PAD - E
    row2d = jnp.concatenate(
        [edge_index[0], jnp.full((pad_e,), N, jnp.int32)]).reshape(E_PAD // G, G)
    col2d = jnp.concatenate(
        [edge_index[1], jnp.full((pad_e,), N, jnp.int32)]).reshape(E_PAD // G, G)

    zeros = jnp.zeros((N_PAD, H), jnp.float32)
    ones_blk = jnp.ones((G, H), jnp.float32)
    x8 = x.reshape(N // 8, 8 * D)
    eye8 = jnp.eye(8, dtype=jnp.float32)
    w1big = jnp.kron(eye8, W1.T)
    w2big = jnp.kron(eye8, W2.T)
    bsum = jnp.kron(eye8, jnp.ones((H, H), jnp.float32))
    b1big = jnp.tile(b1, 8).reshape(1, 8 * H)
    b2big = jnp.tile(b2, 8).reshape(1, 8 * H)

    sc_degree = _make_sc_pass(False)
    sc_propagate = _make_sc_pass(True)

    degp = sc_degree(row2d, col2d, ones_blk, zeros)
    degp8 = degp.reshape(NC, NP8, 8 * H)
    hs1, dinv = _tc1(x8, w1big, b1big, degp8, degp8)
    acc1 = sc_propagate(row2d, col2d, hs1.reshape(N_PAD, H), zeros)
    acc1_8 = acc1.reshape(NC, NP8, 8 * H)
    hs2 = _tc2(acc1_8, acc1_8, hs1, dinv, w2big, b2big)
    acc2 = sc_propagate(row2d, col2d, hs2.reshape(N_PAD, H), zeros)
    acc2_8 = acc2.reshape(NC, NP8, 8 * H)
    out = _tc3(acc2_8, acc2_8, hs2, dinv, bsum)
    return out.reshape(N_PAD, H)[:N]
